# Initial kernel scaffold; baseline (speedup 1.0000x reference)
import jax
import jax.numpy as jnp
from jax.experimental import pallas as pl


def kernel(triangles, face_normals, points):
    raise NotImplementedError("write your pallas kernel here")


# speedup vs baseline: 12.8364x; 12.8364x over previous
import jax
import jax.numpy as jnp
from jax import lax
from jax.experimental import pallas as pl
from jax.experimental.pallas import tpu as pltpu
from jax.experimental.pallas import tpu_sc as plsc

_EPS = 1e-12
P = 128
T = 256
NT = 4096
NP = 8192
BIG = 2 ** 30
NW = 32
PPW = NP // NW
L = 16


def _safe_div(x, y):
    y = jnp.where(jnp.abs(y) < _EPS, _EPS, y)
    return x / y


def _bary_parts(px, py, pz, ax, ay, az, bx, by, bz, cx, cy, cz):
    abx = bx - ax
    aby = by - ay
    abz = bz - az
    acx = cx - ax
    acy = cy - ay
    acz = cz - az

    apx = px - ax
    apy = py - ay
    apz = pz - az
    d1 = abx * apx + aby * apy + abz * apz
    d2_ = acx * apx + acy * apy + acz * apz

    bpx = px - bx
    bpy = py - by
    bpz = pz - bz
    d3 = abx * bpx + aby * bpy + abz * bpz
    d4 = acx * bpx + acy * bpy + acz * bpz

    cpx_ = px - cx
    cpy_ = py - cy
    cpz_ = pz - cz
    d5 = abx * cpx_ + aby * cpy_ + abz * cpz_
    d6 = acx * cpx_ + acy * cpy_ + acz * cpz_

    va = d3 * d6 - d5 * d4
    vb = d5 * d2_ - d1 * d6
    vc = d1 * d4 - d3 * d2_
    v_ab = _safe_div(d1, d1 - d3)
    w_ac = _safe_div(d2_, d2_ - d6)
    w_bc = _safe_div(d4 - d3, (d4 - d3) + (d5 - d6))
    denom = _safe_div(jnp.ones_like(va), va + vb + vc)
    v_in = vb * denom
    w_in = vc * denom
    z = jnp.zeros_like(d1)
    o = jnp.ones_like(d1)

    b0 = 1.0 - v_in - w_in
    b1 = v_in
    b2 = w_in
    cond_bc = (va <= 0) & ((d4 - d3) >= 0) & ((d5 - d6) >= 0)
    b0 = jnp.where(cond_bc, z, b0)
    b1 = jnp.where(cond_bc, 1.0 - w_bc, b1)
    b2 = jnp.where(cond_bc, w_bc, b2)
    cond_ac = (vb <= 0) & (d2_ >= 0) & (d6 <= 0)
    b0 = jnp.where(cond_ac, 1.0 - w_ac, b0)
    b1 = jnp.where(cond_ac, z, b1)
    b2 = jnp.where(cond_ac, w_ac, b2)
    cond_ab = (vc <= 0) & (d1 >= 0) & (d3 <= 0)
    b0 = jnp.where(cond_ab, 1.0 - v_ab, b0)
    b1 = jnp.where(cond_ab, v_ab, b1)
    b2 = jnp.where(cond_ab, z, b2)
    cond_c = (d6 >= 0) & (d5 <= d6)
    b0 = jnp.where(cond_c, z, b0)
    b1 = jnp.where(cond_c, z, b1)
    b2 = jnp.where(cond_c, o, b2)
    cond_b = (d3 >= 0) & (d4 <= d3)
    b0 = jnp.where(cond_b, z, b0)
    b1 = jnp.where(cond_b, o, b1)
    b2 = jnp.where(cond_b, z, b2)
    cond_a = (d1 <= 0) & (d2_ <= 0)
    b0 = jnp.where(cond_a, o, b0)
    b1 = jnp.where(cond_a, z, b1)
    b2 = jnp.where(cond_a, z, b2)
    return b0, b1, b2


def _tc_kernel(tri_ref, pts_ref, faces_ref):
    px = pts_ref[:, 0:1]
    py = pts_ref[:, 1:2]
    pz = pts_ref[:, 2:3]

    nchunks = NT // T
    best_d2 = None
    best_chunk = None

    for c in range(nchunks):
        s = pl.ds(c * T, T)
        ax = tri_ref[0:1, s]
        ay = tri_ref[1:2, s]
        az = tri_ref[2:3, s]
        bx = tri_ref[3:4, s]
        by = tri_ref[4:5, s]
        bz = tri_ref[5:6, s]
        cx = tri_ref[6:7, s]
        cy = tri_ref[7:8, s]
        cz = tri_ref[8:9, s]

        b0, b1, b2 = _bary_parts(px, py, pz, ax, ay, az, bx, by, bz,
                                 cx, cy, cz)

        clx = b0 * ax + b1 * bx + b2 * cx
        cly = b0 * ay + b1 * by + b2 * cy
        clz = b0 * az + b1 * bz + b2 * cz
        dx = clx - px
        dy = cly - py
        dz = clz - pz
        d2p = dx * dx + dy * dy + dz * dz

        if c == 0:
            best_d2 = d2p
            best_chunk = jnp.zeros(d2p.shape, jnp.int32)
        else:
            upd = d2p < best_d2
            best_d2 = jnp.where(upd, d2p, best_d2)
            best_chunk = jnp.where(upd, jnp.int32(c), best_chunk)

    gmin = jnp.min(best_d2, axis=1, keepdims=True)
    lane = lax.broadcasted_iota(jnp.int32, (P, T), 1)
    cand = jnp.where(best_d2 == gmin, best_chunk * T + lane, BIG)
    faces_ref[:, :] = jnp.min(cand, axis=1, keepdims=True)


def _newton_sqrt(a):
    i = lax.bitcast_convert_type(a, jnp.int32)
    s = lax.bitcast_convert_type(
        (i >> 1) + jnp.int32(0x1FBD1DF5), jnp.float32)
    for _ in range(3):
        s = 0.5 * (s + a / s)
    return s


def _sc_body(table_hbm, idx_hbm, px_hbm, py_hbm, pz_hbm,
             sd_hbm, rnx_hbm, rny_hbm, rnz_hbm,
             cpx_hbm, cpy_hbm, cpz_hbm, b0_hbm, b1_hbm, b2_hbm,
             idx_v, table_v, pxv, pyv, pzv,
             sdv, rnxv, rnyv, rnzv, cpxv, cpyv, cpzv, b0v, b1v, b2v):
    wid = lax.axis_index("s") * 2 + lax.axis_index("c")
    base = wid * PPW
    sl = pl.ds(base, PPW)
    pltpu.sync_copy(idx_hbm.at[sl], idx_v)
    pltpu.sync_copy(table_hbm, table_v)
    pltpu.sync_copy(px_hbm.at[sl], pxv)
    pltpu.sync_copy(py_hbm.at[sl], pyv)
    pltpu.sync_copy(pz_hbm.at[sl], pzv)

    for g in range(PPW // L):
        gs = pl.ds(g * L, L)
        flat = idx_v[gs] * 16

        def col(r):
            return plsc.load_gather(table_v, [flat + r])

        axv, ayv, azv = col(0), col(1), col(2)
        bxv, byv, bzv = col(3), col(4), col(5)
        cxv, cyv, czv = col(6), col(7), col(8)
        nxv, nyv, nzv = col(9), col(10), col(11)
        px = pxv[gs]
        py = pyv[gs]
        pz = pzv[gs]

        b0, b1, b2 = _bary_parts(px, py, pz, axv, ayv, azv,
                                 bxv, byv, bzv, cxv, cyv, czv)
        bc0 = jnp.clip(b0, 0.0, 1.0)
        bc1 = jnp.clip(b1, 0.0, 1.0)
        bc2 = jnp.clip(b2, 0.0, 1.0)

        cpx = axv * bc0 + bxv * bc1 + cxv * bc2
        cpy = ayv * bc0 + byv * bc1 + cyv * bc2
        cpz = azv * bc0 + bzv * bc1 + czv * bc2
        rx = cpx - px
        ry = cpy - py
        rz = cpz - pz
        a2 = rx * rx + ry * ry + rz * rz
        norm = _newton_sqrt(a2)
        normc = jnp.where(norm == 0.0, 1.0, norm)
        rnx = rx / normc
        rny = ry / normc
        rnz = rz / normc
        dot = rnx * nxv + rny * nyv + rnz * nzv
        sign = jnp.where(dot > 0.0, -1.0, 1.0)

        sdv[gs] = sign * norm
        rnxv[gs] = rnx
        rnyv[gs] = rny
        rnzv[gs] = rnz
        cpxv[gs] = cpx
        cpyv[gs] = cpy
        cpzv[gs] = cpz
        b0v[gs] = bc0
        b1v[gs] = bc1
        b2v[gs] = bc2

    pltpu.sync_copy(sdv, sd_hbm.at[sl])
    pltpu.sync_copy(rnxv, rnx_hbm.at[sl])
    pltpu.sync_copy(rnyv, rny_hbm.at[sl])
    pltpu.sync_copy(rnzv, rnz_hbm.at[sl])
    pltpu.sync_copy(cpxv, cpx_hbm.at[sl])
    pltpu.sync_copy(cpyv, cpy_hbm.at[sl])
    pltpu.sync_copy(cpzv, cpz_hbm.at[sl])
    pltpu.sync_copy(b0v, b0_hbm.at[sl])
    pltpu.sync_copy(b1v, b1_hbm.at[sl])
    pltpu.sync_copy(b2v, b2_hbm.at[sl])


@jax.jit
def kernel(triangles, face_normals, points):
    tri = triangles.reshape(NT, 9)
    nrm = face_normals.reshape(NT, 3)
    table = jnp.concatenate(
        [tri, nrm, jnp.zeros((NT, 4), jnp.float32)], axis=1).reshape(NT * 16)
    tri_soa = jnp.concatenate(
        [tri.T, jnp.zeros((7, NT), jnp.float32)], axis=0)
    pts = points.reshape(NP, 3)

    faces = pl.pallas_call(
        _tc_kernel,
        grid=(NP // P,),
        in_specs=[
            pl.BlockSpec((16, NT), lambda i: (0, 0)),
            pl.BlockSpec((P, 3), lambda i: (i, 0)),
        ],
        out_specs=pl.BlockSpec((P, 1), lambda i: (i, 0)),
        out_shape=jax.ShapeDtypeStruct((NP, 1), jnp.int32),
    )(tri_soa, pts)

    idx_flat = faces.reshape(NP)
    px = pts[:, 0]
    py = pts[:, 1]
    pz = pts[:, 2]

    sc = pl.kernel(
        _sc_body,
        out_type=[jax.ShapeDtypeStruct((NP,), jnp.float32)] * 10,
        mesh=plsc.VectorSubcoreMesh(core_axis_name="c", subcore_axis_name="s"),
        compiler_params=pltpu.CompilerParams(needs_layout_passes=False),
        scratch_types=(
            [pltpu.VMEM((PPW,), jnp.int32),
             pltpu.VMEM((NT * 16,), jnp.float32)]
            + [pltpu.VMEM((PPW,), jnp.float32)] * 13
        ),
    )
    (sd, rnx, rny, rnz, cpx, cpy, cpz, b0, b1, b2) = sc(
        table, idx_flat, px, py, pz)

    rn = jnp.stack([rnx, rny, rnz], axis=-1)
    cp = jnp.stack([cpx, cpy, cpz], axis=-1)
    bcs = jnp.stack([b0, b1, b2], axis=-1)
    return (sd.reshape(1, NP), rn.reshape(1, NP, 3), cp.reshape(1, NP, 3),
            idx_flat.reshape(1, NP), bcs.reshape(1, NP, 3))


# speedup vs baseline: 15.0392x; 1.1716x over previous
import jax
import jax.numpy as jnp
from jax import lax
from jax.experimental import pallas as pl
from jax.experimental.pallas import tpu as pltpu
from jax.experimental.pallas import tpu_sc as plsc

_EPS = 1e-12
P = 128
T = 256
NT = 4096
NP = 8192
BIG = 2 ** 30
NW = 32
PPW = NP // NW
L = 16


def _safe_div(x, y):
    y = jnp.where(jnp.abs(y) < _EPS, _EPS, y)
    return x / y


def _bary_parts(px, py, pz, ax, ay, az, bx, by, bz, cx, cy, cz):
    abx = bx - ax
    aby = by - ay
    abz = bz - az
    acx = cx - ax
    acy = cy - ay
    acz = cz - az

    apx = px - ax
    apy = py - ay
    apz = pz - az
    d1 = abx * apx + aby * apy + abz * apz
    d2_ = acx * apx + acy * apy + acz * apz

    bpx = px - bx
    bpy = py - by
    bpz = pz - bz
    d3 = abx * bpx + aby * bpy + abz * bpz
    d4 = acx * bpx + acy * bpy + acz * bpz

    cpx_ = px - cx
    cpy_ = py - cy
    cpz_ = pz - cz
    d5 = abx * cpx_ + aby * cpy_ + abz * cpz_
    d6 = acx * cpx_ + acy * cpy_ + acz * cpz_

    va = d3 * d6 - d5 * d4
    vb = d5 * d2_ - d1 * d6
    vc = d1 * d4 - d3 * d2_
    v_ab = _safe_div(d1, d1 - d3)
    w_ac = _safe_div(d2_, d2_ - d6)
    w_bc = _safe_div(d4 - d3, (d4 - d3) + (d5 - d6))
    denom = _safe_div(jnp.ones_like(va), va + vb + vc)
    v_in = vb * denom
    w_in = vc * denom
    z = jnp.zeros_like(d1)
    o = jnp.ones_like(d1)

    b0 = 1.0 - v_in - w_in
    b1 = v_in
    b2 = w_in
    cond_bc = (va <= 0) & ((d4 - d3) >= 0) & ((d5 - d6) >= 0)
    b0 = jnp.where(cond_bc, z, b0)
    b1 = jnp.where(cond_bc, 1.0 - w_bc, b1)
    b2 = jnp.where(cond_bc, w_bc, b2)
    cond_ac = (vb <= 0) & (d2_ >= 0) & (d6 <= 0)
    b0 = jnp.where(cond_ac, 1.0 - w_ac, b0)
    b1 = jnp.where(cond_ac, z, b1)
    b2 = jnp.where(cond_ac, w_ac, b2)
    cond_ab = (vc <= 0) & (d1 >= 0) & (d3 <= 0)
    b0 = jnp.where(cond_ab, 1.0 - v_ab, b0)
    b1 = jnp.where(cond_ab, v_ab, b1)
    b2 = jnp.where(cond_ab, z, b2)
    cond_c = (d6 >= 0) & (d5 <= d6)
    b0 = jnp.where(cond_c, z, b0)
    b1 = jnp.where(cond_c, z, b1)
    b2 = jnp.where(cond_c, o, b2)
    cond_b = (d3 >= 0) & (d4 <= d3)
    b0 = jnp.where(cond_b, z, b0)
    b1 = jnp.where(cond_b, o, b1)
    b2 = jnp.where(cond_b, z, b2)
    cond_a = (d1 <= 0) & (d2_ <= 0)
    b0 = jnp.where(cond_a, o, b0)
    b1 = jnp.where(cond_a, z, b1)
    b2 = jnp.where(cond_a, z, b2)
    return b0, b1, b2


def _tc_kernel(tri_ref, pts_ref, faces_ref):
    px = pts_ref[:, 0:1]
    py = pts_ref[:, 1:2]
    pz = pts_ref[:, 2:3]

    nchunks = NT // T
    best_d2 = None
    best_chunk = None

    for c in range(nchunks):
        s = pl.ds(c * T, T)
        ax = tri_ref[0:1, s]
        ay = tri_ref[1:2, s]
        az = tri_ref[2:3, s]
        bx = tri_ref[3:4, s]
        by = tri_ref[4:5, s]
        bz = tri_ref[5:6, s]
        cx = tri_ref[6:7, s]
        cy = tri_ref[7:8, s]
        cz = tri_ref[8:9, s]

        b0, b1, b2 = _bary_parts(px, py, pz, ax, ay, az, bx, by, bz,
                                 cx, cy, cz)

        clx = b0 * ax + b1 * bx + b2 * cx
        cly = b0 * ay + b1 * by + b2 * cy
        clz = b0 * az + b1 * bz + b2 * cz
        dx = clx - px
        dy = cly - py
        dz = clz - pz
        d2p = dx * dx + dy * dy + dz * dz

        if c == 0:
            best_d2 = d2p
            best_chunk = jnp.zeros(d2p.shape, jnp.int32)
        else:
            upd = d2p < best_d2
            best_d2 = jnp.where(upd, d2p, best_d2)
            best_chunk = jnp.where(upd, jnp.int32(c), best_chunk)

    gmin = jnp.min(best_d2, axis=1, keepdims=True)
    lane = lax.broadcasted_iota(jnp.int32, (P, T), 1)
    cand = jnp.where(best_d2 == gmin, best_chunk * T + lane, BIG)
    faces_ref[:, :] = jnp.min(cand, axis=1, keepdims=True)


NTC = 6144
NSC = NP - NTC
SPT = NSC // NW


def _sc_score_body(table_hbm, px_hbm, py_hbm, pz_hbm, faces_hbm,
                   table_v, pxv, pyv, pzv, faces_v):
    wid = lax.axis_index("s") * 2 + lax.axis_index("c")
    base = wid * SPT
    sl = pl.ds(NTC + base, SPT)
    pltpu.sync_copy(table_hbm, table_v)
    pltpu.sync_copy(px_hbm.at[sl], pxv)
    pltpu.sync_copy(py_hbm.at[sl], pyv)
    pltpu.sync_copy(pz_hbm.at[sl], pzv)

    iota = lax.iota(jnp.int32, L)
    zmask = iota == 0

    def point_body(p, _):
        pidx = jnp.full((L,), p, jnp.int32)
        px = plsc.load_gather(pxv, [pidx])
        py = plsc.load_gather(pyv, [pidx])
        pz = plsc.load_gather(pzv, [pidx])

        def chunk_body(c, carry):
            best_d2, best_chunk = carry
            off = c * L

            def row(r):
                return plsc.load_gather(table_v, [iota + (r * NT + off)])

            ax, ay, az = row(0), row(1), row(2)
            bx, by, bz = row(3), row(4), row(5)
            cx, cy, cz = row(6), row(7), row(8)
            b0, b1, b2 = _bary_parts(px, py, pz, ax, ay, az,
                                     bx, by, bz, cx, cy, cz)
            clx = b0 * ax + b1 * bx + b2 * cx
            cly = b0 * ay + b1 * by + b2 * cy
            clz = b0 * az + b1 * bz + b2 * cz
            dx = clx - px
            dy = cly - py
            dz = clz - pz
            d2p = dx * dx + dy * dy + dz * dz
            upd = d2p < best_d2
            best_d2 = jnp.where(upd, d2p, best_d2)
            best_chunk = jnp.where(upd, c, best_chunk)
            return best_d2, best_chunk

        init = (jnp.full((L,), jnp.inf, jnp.float32),
                jnp.zeros((L,), jnp.int32))
        best_d2, best_chunk = lax.fori_loop(0, NT // L, chunk_body, init)

        gmin = lax.reduce_min(best_d2, axes=(0,))
        cand = jnp.where(best_d2 == gmin, best_chunk * L + iota, BIG)
        idx = lax.reduce_min(cand, axes=(0,))
        plsc.store_scatter(faces_v, [pidx], jnp.full((L,), idx, jnp.int32),
                           mask=zmask)
        return 0

    lax.fori_loop(0, SPT, point_body, 0)
    pltpu.sync_copy(faces_v, faces_hbm.at[pl.ds(base, SPT)])


def _newton_sqrt(a):
    i = lax.bitcast_convert_type(a, jnp.int32)
    s = lax.bitcast_convert_type(
        (i >> 1) + jnp.int32(0x1FBD1DF5), jnp.float32)
    for _ in range(3):
        s = 0.5 * (s + a / s)
    return s


def _sc_body(table_hbm, idx_hbm, px_hbm, py_hbm, pz_hbm,
             sd_hbm, rnx_hbm, rny_hbm, rnz_hbm,
             cpx_hbm, cpy_hbm, cpz_hbm, b0_hbm, b1_hbm, b2_hbm,
             idx_v, table_v, pxv, pyv, pzv,
             sdv, rnxv, rnyv, rnzv, cpxv, cpyv, cpzv, b0v, b1v, b2v):
    wid = lax.axis_index("s") * 2 + lax.axis_index("c")
    base = wid * PPW
    sl = pl.ds(base, PPW)
    pltpu.sync_copy(idx_hbm.at[sl], idx_v)
    pltpu.sync_copy(table_hbm, table_v)
    pltpu.sync_copy(px_hbm.at[sl], pxv)
    pltpu.sync_copy(py_hbm.at[sl], pyv)
    pltpu.sync_copy(pz_hbm.at[sl], pzv)

    for g in range(PPW // L):
        gs = pl.ds(g * L, L)
        flat = idx_v[gs] * 16

        def col(r):
            return plsc.load_gather(table_v, [flat + r])

        axv, ayv, azv = col(0), col(1), col(2)
        bxv, byv, bzv = col(3), col(4), col(5)
        cxv, cyv, czv = col(6), col(7), col(8)
        nxv, nyv, nzv = col(9), col(10), col(11)
        px = pxv[gs]
        py = pyv[gs]
        pz = pzv[gs]

        b0, b1, b2 = _bary_parts(px, py, pz, axv, ayv, azv,
                                 bxv, byv, bzv, cxv, cyv, czv)
        bc0 = jnp.clip(b0, 0.0, 1.0)
        bc1 = jnp.clip(b1, 0.0, 1.0)
        bc2 = jnp.clip(b2, 0.0, 1.0)

        cpx = axv * bc0 + bxv * bc1 + cxv * bc2
        cpy = ayv * bc0 + byv * bc1 + cyv * bc2
        cpz = azv * bc0 + bzv * bc1 + czv * bc2
        rx = cpx - px
        ry = cpy - py
        rz = cpz - pz
        a2 = rx * rx + ry * ry + rz * rz
        norm = _newton_sqrt(a2)
        normc = jnp.where(norm == 0.0, 1.0, norm)
        rnx = rx / normc
        rny = ry / normc
        rnz = rz / normc
        dot = rnx * nxv + rny * nyv + rnz * nzv
        sign = jnp.where(dot > 0.0, -1.0, 1.0)

        sdv[gs] = sign * norm
        rnxv[gs] = rnx
        rnyv[gs] = rny
        rnzv[gs] = rnz
        cpxv[gs] = cpx
        cpyv[gs] = cpy
        cpzv[gs] = cpz
        b0v[gs] = bc0
        b1v[gs] = bc1
        b2v[gs] = bc2

    pltpu.sync_copy(sdv, sd_hbm.at[sl])
    pltpu.sync_copy(rnxv, rnx_hbm.at[sl])
    pltpu.sync_copy(rnyv, rny_hbm.at[sl])
    pltpu.sync_copy(rnzv, rnz_hbm.at[sl])
    pltpu.sync_copy(cpxv, cpx_hbm.at[sl])
    pltpu.sync_copy(cpyv, cpy_hbm.at[sl])
    pltpu.sync_copy(cpzv, cpz_hbm.at[sl])
    pltpu.sync_copy(b0v, b0_hbm.at[sl])
    pltpu.sync_copy(b1v, b1_hbm.at[sl])
    pltpu.sync_copy(b2v, b2_hbm.at[sl])


@jax.jit
def kernel(triangles, face_normals, points):
    tri = triangles.reshape(NT, 9)
    nrm = face_normals.reshape(NT, 3)
    table = jnp.concatenate(
        [tri, nrm, jnp.zeros((NT, 4), jnp.float32)], axis=1).reshape(NT * 16)
    tri_soa = jnp.concatenate(
        [tri.T, jnp.zeros((7, NT), jnp.float32)], axis=0)
    pts = points.reshape(NP, 3)

    px = pts[:, 0]
    py = pts[:, 1]
    pz = pts[:, 2]
    table_soa = tri.T.reshape(9 * NT)

    faces_tc = pl.pallas_call(
        _tc_kernel,
        grid=(NTC // P,),
        in_specs=[
            pl.BlockSpec((16, NT), lambda i: (0, 0)),
            pl.BlockSpec((P, 3), lambda i: (i, 0)),
        ],
        out_specs=pl.BlockSpec((P, 1), lambda i: (i, 0)),
        out_shape=jax.ShapeDtypeStruct((NTC, 1), jnp.int32),
    )(tri_soa, pts)

    sc_score = pl.kernel(
        _sc_score_body,
        out_type=jax.ShapeDtypeStruct((NSC,), jnp.int32),
        mesh=plsc.VectorSubcoreMesh(core_axis_name="c", subcore_axis_name="s"),
        compiler_params=pltpu.CompilerParams(needs_layout_passes=False),
        scratch_types=[
            pltpu.VMEM((9 * NT,), jnp.float32),
            pltpu.VMEM((SPT,), jnp.float32),
            pltpu.VMEM((SPT,), jnp.float32),
            pltpu.VMEM((SPT,), jnp.float32),
            pltpu.VMEM((SPT,), jnp.int32),
        ],
    )
    faces_sc = sc_score(table_soa, px, py, pz)

    idx_flat = jnp.concatenate([faces_tc.reshape(NTC), faces_sc])

    sc = pl.kernel(
        _sc_body,
        out_type=[jax.ShapeDtypeStruct((NP,), jnp.float32)] * 10,
        mesh=plsc.VectorSubcoreMesh(core_axis_name="c", subcore_axis_name="s"),
        compiler_params=pltpu.CompilerParams(needs_layout_passes=False),
        scratch_types=(
            [pltpu.VMEM((PPW,), jnp.int32),
             pltpu.VMEM((NT * 16,), jnp.float32)]
            + [pltpu.VMEM((PPW,), jnp.float32)] * 13
        ),
    )
    (sd, rnx, rny, rnz, cpx, cpy, cpz, b0, b1, b2) = sc(
        table, idx_flat, px, py, pz)

    rn = jnp.stack([rnx, rny, rnz], axis=-1)
    cp = jnp.stack([cpx, cpy, cpz], axis=-1)
    bcs = jnp.stack([b0, b1, b2], axis=-1)
    return (sd.reshape(1, NP), rn.reshape(1, NP, 3), cp.reshape(1, NP, 3),
            idx_flat.reshape(1, NP), bcs.reshape(1, NP, 3))
